# Initial kernel scaffold; baseline (speedup 1.0000x reference)
import jax
import jax.numpy as jnp
from jax.experimental import pallas as pl


def kernel(z):
    raise NotImplementedError("write your pallas kernel here")


# speedup vs baseline: 1.1416x; 1.1416x over previous
import functools

import jax
import jax.numpy as jnp
import numpy as np
from jax import lax
from jax.experimental import pallas as pl
from jax.experimental.pallas import tpu as pltpu
from jax.experimental.pallas import tpu_sc as plsc

_LEVELS = [8, 8, 8, 5, 5, 5]
_D = 6
_B, _S = 256, 1024
_N = _B * _S
_CB = int(np.prod(_LEVELS))
_NC, _NS = 2, 16
_NW = _NC * _NS
_ROWS_PER_BLK = 64

_BASES = [int(np.prod(_LEVELS[i + 1:])) for i in range(_D - 1)] + [1]


def _tc_flat_body(z_ref, flat_ref):
    acc = None
    for k in range(_D):
        lvl = _LEVELS[k]
        half = float(lvl // 2)
        x = jnp.tanh(z_ref[k]) * half
        q = jnp.round(x)
        digit = jnp.minimum(jnp.maximum(q + half, 0.0), float(lvl - 1))
        term = digit * float(_BASES[k])
        acc = term if acc is None else acc + term
    flat_ref[...] = acc.astype(jnp.int32)


def _tc_flat(zp):
    rows = 128
    grid = (_B // rows,)
    slab_spec = pl.BlockSpec((_D, rows, _S), lambda i: (0, i, 0))
    flat_spec = pl.BlockSpec((rows, _S), lambda i: (i, 0))
    return pl.pallas_call(
        _tc_flat_body,
        grid=grid,
        in_specs=[slab_spec],
        out_specs=flat_spec,
        out_shape=jax.ShapeDtypeStruct((_B, _S), jnp.int32),
    )(zp)


def _tc_zq_body(z_ref, zq_ref):
    for k in range(_D):
        lvl = _LEVELS[k]
        half = float(lvl // 2)
        x = jnp.tanh(z_ref[k]) * half
        zq_ref[k] = jnp.round(x) * (1.0 / lvl)


def _tc_zq(zp):
    grid = (_B // _ROWS_PER_BLK,)
    slab_spec = pl.BlockSpec((_D, _ROWS_PER_BLK, _S), lambda i: (0, i, 0))
    return pl.pallas_call(
        _tc_zq_body,
        grid=grid,
        in_specs=[slab_spec],
        out_specs=slab_spec,
        out_shape=jax.ShapeDtypeStruct((_D, _B, _S), jnp.float32),
    )(zp)


@functools.cache
def _sc_hist_kernel():
    return pl.kernel(
        _sc_hist_body,
        out_type=jax.ShapeDtypeStruct((_NC, _CB), jnp.float32),
        mesh=plsc.VectorSubcoreMesh(core_axis_name="c", subcore_axis_name="s",
                                    num_cores=_NC, num_subcores=_NS),
        compiler_params=pltpu.CompilerParams(needs_layout_passes=False),
        scratch_types=[
            pltpu.VMEM((64, 128), jnp.int32),
            pltpu.VMEM((128,), jnp.float32),
            pltpu.VMEM_SHARED((_CB,), jnp.float32),
            pltpu.SemaphoreType.DMA,
        ],
    )


def _sc_hist_body(idx_hbm, zeros_hbm, hist_hbm, idx_v, ones_v, hist_sh, sem):
    c = lax.axis_index("c")
    s = lax.axis_index("s")
    wid = s * _NC + c

    @pl.when(s == 0)
    def _zero():
        pltpu.sync_copy(zeros_hbm, hist_sh)

    pltpu.sync_copy(idx_hbm.at[wid], idx_v)

    ones16 = jnp.full((16,), 1.0, jnp.float32)
    for k in range(8):
        ones_v[pl.ds(k * 16, 16)] = ones16

    plsc.subcore_barrier()

    def fire(j, carry):
        for k in range(16):
            pltpu.async_copy(ones_v, hist_sh.at[idx_v.at[j * 16 + k]],
                             sem, add=True)
        return carry

    lax.fori_loop(0, 4, fire, 0)

    def drain(j, carry):
        for k in range(16):
            pltpu.make_async_copy(ones_v, hist_sh.at[idx_v.at[j * 16 + k]],
                                  sem).wait()
        return carry

    lax.fori_loop(0, 4, drain, 0)

    plsc.subcore_barrier()

    @pl.when(s == 0)
    def _flush():
        pltpu.sync_copy(hist_sh, hist_hbm.at[c])


def _tc_ppl_body(hist_ref, out_ref):
    h = hist_ref[...]
    tot = h[0:1, :] + h[1:2, :]
    e = tot * (1.0 / _N)
    ent = jnp.sum(e * jnp.log(e + 1e-8))
    out_ref[...] = jnp.exp(-ent) * jnp.ones((1, 1), jnp.float32)


def _tc_ppl(hist):
    return pl.pallas_call(
        _tc_ppl_body,
        out_shape=jax.ShapeDtypeStruct((1, 1), jnp.float32),
    )(hist)


def kernel(z):
    zp = z.transpose(2, 0, 1)
    flat = _tc_flat(zp)
    zeros = jnp.zeros((_CB,), jnp.float32)
    hist = _sc_hist_kernel()(flat.reshape(_NW, 64, 128), zeros)
    zqp = _tc_zq(zp)
    perplexity = _tc_ppl(hist)[0, 0]
    z_q = zqp.transpose(1, 2, 0)
    return (z_q, flat, perplexity)
